# baseline (speedup 1.0000x reference)
import functools

import jax
import jax.numpy as jnp
from jax import lax
from jax.experimental import pallas as pl

K = 3
N = 4096
NS = 16384
D = 256
DS = 128
H = 256
B = 256


def _body(posq, xs, posct, xc, w1h, w1s, b1, w2, b2, out):
    q = posq[:]
    ct = posct[:]

    qn = jnp.sum(q * q, axis=1, keepdims=True)
    cn = jnp.sum(ct * ct, axis=0, keepdims=True)
    qc = lax.dot_general(q.astype(jnp.bfloat16), ct.astype(jnp.bfloat16),
                         (((1,), (0,)), ((), ())),
                         preferred_element_type=jnp.float32)
    d2 = jnp.maximum(qn + cn - 2.0 * qc, 0.0)

    inf = jnp.float32(jnp.inf)

    d2m = d2
    S = jnp.zeros((B, N), jnp.float32)
    wsum = jnp.zeros((B, 1), jnp.float32)
    for _ in range(K):
        m = jnp.min(d2m, axis=1, keepdims=True)
        eqb = d2m == m
        eqf = eqb.astype(jnp.float32)
        wk = (1.0 / jnp.maximum(m, 1e-16)).astype(jnp.bfloat16)
        S = S + wk.astype(jnp.float32) * eqf
        wsum = wsum + wk.astype(jnp.float32)
        d2m = jnp.where(eqb, inf, d2m)

    cnt = jnp.sum((d2m == inf).astype(jnp.float32), axis=1, keepdims=True)
    bad = jnp.max(cnt) != 3.0

    def _exact():
        iota = lax.broadcasted_iota(jnp.int32, (B, N), 1)
        big = jnp.int32(2**30)
        dd = d2
        Se = jnp.zeros((B, N), jnp.float32)
        we = jnp.zeros((B, 1), jnp.float32)
        for _ in range(K):
            me = jnp.min(dd, axis=1, keepdims=True)
            j = jnp.min(jnp.where(dd == me, iota, big), axis=1, keepdims=True)
            sel = iota == j
            wke = (1.0 / jnp.maximum(me, 1e-16)).astype(jnp.bfloat16)
            Se = jnp.where(sel, wke.astype(jnp.float32), Se)
            we = we + wke.astype(jnp.float32)
            dd = jnp.where(sel, inf, dd)
        return Se, we

    S, wsum = lax.cond(bad, _exact, lambda: (S, wsum))

    h = lax.dot_general(S.astype(jnp.bfloat16), xc[:],
                        (((1,), (0,)), ((), ())),
                        preferred_element_type=jnp.float32) / wsum

    a = (lax.dot_general(h.astype(jnp.bfloat16), w1h[:],
                         (((1,), (0,)), ((), ())),
                         preferred_element_type=jnp.float32)
         + lax.dot_general(xs[:], w1s[:], (((1,), (0,)), ((), ())),
                           preferred_element_type=jnp.float32)
         + b1[:])
    a = jnp.maximum(a, 0.0)
    out[:] = lax.dot_general(a.astype(jnp.bfloat16), w2[:],
                             (((1,), (0,)), ((), ())),
                             preferred_element_type=jnp.float32) + b2[:]


@jax.jit
def _run(posq_pad, posct_pad, xb, xsb, W1h, W1s, b1, W2, b2):
    grid = (NS // B,)
    return pl.pallas_call(
        _body,
        grid=grid,
        in_specs=[
            pl.BlockSpec((B, 8), lambda i: (i, 0)),
            pl.BlockSpec((B, DS), lambda i: (i, 0)),
            pl.BlockSpec((8, N), lambda i: (0, 0)),
            pl.BlockSpec((N, D), lambda i: (0, 0)),
            pl.BlockSpec((D, H), lambda i: (0, 0)),
            pl.BlockSpec((DS, H), lambda i: (0, 0)),
            pl.BlockSpec((1, H), lambda i: (0, 0)),
            pl.BlockSpec((H, H), lambda i: (0, 0)),
            pl.BlockSpec((1, H), lambda i: (0, 0)),
        ],
        out_specs=pl.BlockSpec((B, H), lambda i: (i, 0)),
        out_shape=jax.ShapeDtypeStruct((NS, H), jnp.float32),
    )(posq_pad, xsb, posct_pad, xb, W1h, W1s, b1, W2, b2)


def kernel(x, pos, batch, x_skip, pos_skip, batch_skip, W1, b1, W2, b2):
    posq_pad = jnp.zeros((NS, 8), jnp.float32).at[:, :3].set(pos_skip)
    posct_pad = jnp.zeros((8, N), jnp.float32).at[:3, :].set(pos.T)
    out = _run(posq_pad, posct_pad,
               x.astype(jnp.bfloat16), x_skip.astype(jnp.bfloat16),
               W1[:D].astype(jnp.bfloat16), W1[D:].astype(jnp.bfloat16),
               b1.reshape(1, H), W2.astype(jnp.bfloat16), b2.reshape(1, H))
    return (out, pos_skip, batch_skip)


# speedup vs baseline: 1.9947x; 1.9947x over previous
import functools

import jax
import jax.numpy as jnp
from jax import lax
from jax.experimental import pallas as pl

K = 3
N = 4096
NS = 16384
D = 256
DS = 128
H = 256
B = 512


def _body(posq, xs, posct, xc, w1h, w1s, b1, w2, b2, out):
    q = posq[:]
    ct = posct[:]

    qn = jnp.sum(q * q, axis=1, keepdims=True)
    cn = jnp.sum(ct * ct, axis=0, keepdims=True)
    qc = lax.dot_general(q.astype(jnp.bfloat16), ct.astype(jnp.bfloat16),
                         (((1,), (0,)), ((), ())),
                         preferred_element_type=jnp.float32)
    d2 = jnp.maximum(qn + cn - 2.0 * qc, 0.0)

    inf = jnp.float32(jnp.inf)

    iota = lax.broadcasted_iota(jnp.int32, (B, N), 1)
    big = jnp.int32(2**30)
    S = jnp.zeros((B, N), jnp.float32)
    wsum = jnp.zeros((B, 1), jnp.float32)
    for _ in range(K):
        m = jnp.min(d2, axis=1, keepdims=True)
        j = jnp.min(jnp.where(d2 == m, iota, big), axis=1, keepdims=True)
        sel = iota == j
        wk = (1.0 / jnp.maximum(m, 1e-16)).astype(jnp.bfloat16)
        S = jnp.where(sel, wk.astype(jnp.float32), S)
        wsum = wsum + wk.astype(jnp.float32)
        d2 = jnp.where(sel, inf, d2)

    h = lax.dot_general(S.astype(jnp.bfloat16), xc[:],
                        (((1,), (0,)), ((), ())),
                        preferred_element_type=jnp.float32) / wsum

    a = (lax.dot_general(h.astype(jnp.bfloat16), w1h[:],
                         (((1,), (0,)), ((), ())),
                         preferred_element_type=jnp.float32)
         + lax.dot_general(xs[:], w1s[:], (((1,), (0,)), ((), ())),
                           preferred_element_type=jnp.float32)
         + b1[:])
    a = jnp.maximum(a, 0.0)
    out[:] = lax.dot_general(a.astype(jnp.bfloat16), w2[:],
                             (((1,), (0,)), ((), ())),
                             preferred_element_type=jnp.float32) + b2[:]


@jax.jit
def _run(posq_pad, posct_pad, xb, xsb, W1h, W1s, b1, W2, b2):
    grid = (NS // B,)
    return pl.pallas_call(
        _body,
        grid=grid,
        in_specs=[
            pl.BlockSpec((B, 8), lambda i: (i, 0)),
            pl.BlockSpec((B, DS), lambda i: (i, 0)),
            pl.BlockSpec((8, N), lambda i: (0, 0)),
            pl.BlockSpec((N, D), lambda i: (0, 0)),
            pl.BlockSpec((D, H), lambda i: (0, 0)),
            pl.BlockSpec((DS, H), lambda i: (0, 0)),
            pl.BlockSpec((1, H), lambda i: (0, 0)),
            pl.BlockSpec((H, H), lambda i: (0, 0)),
            pl.BlockSpec((1, H), lambda i: (0, 0)),
        ],
        out_specs=pl.BlockSpec((B, H), lambda i: (i, 0)),
        out_shape=jax.ShapeDtypeStruct((NS, H), jnp.float32),
    )(posq_pad, xsb, posct_pad, xb, W1h, W1s, b1, W2, b2)


def kernel(x, pos, batch, x_skip, pos_skip, batch_skip, W1, b1, W2, b2):
    posq_pad = jnp.zeros((NS, 8), jnp.float32).at[:, :3].set(pos_skip)
    posct_pad = jnp.zeros((8, N), jnp.float32).at[:3, :].set(pos.T)
    out = _run(posq_pad, posct_pad,
               x.astype(jnp.bfloat16), x_skip.astype(jnp.bfloat16),
               W1[:D].astype(jnp.bfloat16), W1[D:].astype(jnp.bfloat16),
               b1.reshape(1, H), W2.astype(jnp.bfloat16), b2.reshape(1, H))
    return (out, pos_skip, batch_skip)
